# baseline (speedup 1.0000x reference)
import dataclasses
import functools

import jax
import jax.numpy as jnp
from jax import lax
from jax.experimental import pallas as pl
from jax.experimental.pallas import tpu as pltpu
from jax.experimental.pallas import tpu_sc as plsc

B = 1024
D = 256
NW = 256
NS = 65536
TOP_K = 64
CHUNK = 128
NCHUNK = NS // CHUNK
MAXCH = 512
CAND = 1024
NTILES = 32
ROWS_PER_TILE = B // NTILES


def _sem_score_kernel(q_ref, k_ref, n_ref, out_ref, m_ref):
    k = k_ref[...]
    kn = k / jnp.maximum(n_ref[...], 1e-6)
    s = lax.dot_general(q_ref[...], kn, (((1,), (1,)), ((), ())),
                        preferred_element_type=jnp.float32)
    out_ref[...] = s
    for j in range(s.shape[1] // CHUNK):
        m_ref[0, :, j:j + 1] = jnp.max(
            s[:, j * CHUNK:(j + 1) * CHUNK], axis=-1, keepdims=True)


def _iota16():
    return lax.broadcasted_iota(jnp.int32, (16,), 0)


def _select_kernel(sem_hbm, ws_hbm, m_hbm, tau_hbm,
                   ci_hbm, cs_hbm, cnt_hbm,
                   m_v, ws_v, tau_v, glist_v, bases_v, fetch_v,
                   ci_v, cs_v, scr_v):
    wid = lax.axis_index("s") * 2 + lax.axis_index("c")

    for j in range(MAXCH // 16):
        glist_v[j // 8, pl.ds((j % 8) * 16, 16)] = jnp.zeros((16,), jnp.int32)

    @pl.loop(0, ROWS_PER_TILE)
    def _row(rr):
        row = wid * ROWS_PER_TILE + rr
        pltpu.sync_copy(tau_hbm.at[row], tau_v)
        pltpu.sync_copy(ws_hbm.at[row], ws_v)
        pltpu.sync_copy(m_hbm.at[row], m_v)
        tau = tau_v[...]

        nc = jnp.zeros((16,), jnp.int32)
        row_base = row * NCHUNK
        for j in range(NCHUNK // 16):
            mv = m_v[pl.ds(16 * j, 16)]
            msk = mv >= tau
            m32 = jnp.where(msk, 1, 0).astype(jnp.int32)
            pos = nc + plsc.cumsum(m32) - m32
            ok = msk & (pos < MAXCH)
            cid = 16 * j + _iota16()
            plsc.store_scatter(glist_v, [pos // CHUNK, pos % CHUNK],
                               row_base + cid, mask=ok)
            plsc.store_scatter(bases_v, [pos], NW + cid * CHUNK, mask=ok)
            nc = nc + plsc.all_reduce_population_count(msk)

        nc_s = jnp.minimum(jnp.max(nc, axis=0), MAXCH)

        nb = (nc_s + 127) // 128

        @pl.loop(0, nb)
        def _batch(bi):
            pltpu.sync_copy(sem_hbm.at[glist_v.at[bi]],
                            fetch_v.at[pl.ds(bi * 128, 128)])

        for j in range(CAND // 16):
            cs_v[pl.ds(16 * j, 16)] = jnp.full((16,), -jnp.inf, jnp.float32)

        cc = jnp.zeros((16,), jnp.int32)
        for j in range(NW // 16):
            s = ws_v[pl.ds(16 * j, 16)]
            msk = s >= tau
            m32 = jnp.where(msk, 1, 0).astype(jnp.int32)
            pos = cc + plsc.cumsum(m32) - m32
            ok = msk & (pos < CAND)
            plsc.store_scatter(cs_v, [pos], s, mask=ok)
            plsc.store_scatter(ci_v, [pos], 16 * j + _iota16(), mask=ok)
            cc = cc + plsc.all_reduce_population_count(msk)

        def chunk_body(i, cc):
            base = plsc.load_gather(bases_v, [jnp.full((16,), i, jnp.int32)])
            for j in range(CHUNK // 16):
                s = fetch_v.at[i][pl.ds(16 * j, 16)]
                msk = s >= tau
                m32 = jnp.where(msk, 1, 0).astype(jnp.int32)
                pos = cc + plsc.cumsum(m32) - m32
                ok = msk & (pos < CAND)
                plsc.store_scatter(cs_v, [pos], s, mask=ok)
                plsc.store_scatter(ci_v, [pos], base + 16 * j + _iota16(), mask=ok)
                cc = cc + plsc.all_reduce_population_count(msk)
            return cc

        cc = lax.fori_loop(0, nc_s, chunk_body, cc)

        scr_v[...] = cc
        pltpu.sync_copy(scr_v, cnt_hbm.at[row])
        pltpu.sync_copy(cs_v, cs_hbm.at[row])
        pltpu.sync_copy(ci_v, ci_hbm.at[row])


def _run_select(sem_chunks, working_scores, m, tau_b):
    mesh = plsc.VectorSubcoreMesh(core_axis_name="c", subcore_axis_name="s")
    cp = pltpu.CompilerParams()
    if "needs_layout_passes" in pltpu.CompilerParams.__dataclass_fields__:
        cp = dataclasses.replace(cp, needs_layout_passes=False)
    sel = functools.partial(
        pl.kernel,
        mesh=mesh,
        compiler_params=cp,
        out_type=[
            jax.ShapeDtypeStruct((B, CAND), jnp.int32),
            jax.ShapeDtypeStruct((B, CAND), jnp.float32),
            jax.ShapeDtypeStruct((B, 16), jnp.int32),
        ],
        scratch_types=[
            pltpu.VMEM((NCHUNK,), jnp.float32),
            pltpu.VMEM((NW,), jnp.float32),
            pltpu.VMEM((16,), jnp.float32),
            pltpu.VMEM((MAXCH // 128, 128), jnp.int32),
            pltpu.VMEM((MAXCH,), jnp.int32),
            pltpu.VMEM((MAXCH, CHUNK), jnp.float32),
            pltpu.VMEM((CAND,), jnp.int32),
            pltpu.VMEM((CAND,), jnp.float32),
            pltpu.VMEM((16,), jnp.int32),
        ],
    )(_select_kernel)
    return sel(sem_chunks, working_scores, m, tau_b)


def kernel(current, previous, working_keys, working_values,
           semantic_keys, semantic_values, W):
    q = jnp.concatenate([current, previous], axis=-1) @ W.T
    q = q / jnp.maximum(jnp.linalg.norm(q, axis=-1, keepdims=True), 1e-6)

    nwk = working_keys / jnp.maximum(
        jnp.linalg.norm(working_keys, axis=-1, keepdims=True), 1e-6)
    working_scores = jnp.einsum('bd,bnd->bn', q, nwk)

    sk_norm = jnp.linalg.norm(semantic_keys, axis=-1, keepdims=True)

    NSB = 512
    semantic_scores, m = pl.pallas_call(
        _sem_score_kernel,
        grid=(NS // NSB,),
        in_specs=[pl.BlockSpec((B, D), lambda i: (0, 0)),
                  pl.BlockSpec((NSB, D), lambda i: (i, 0)),
                  pl.BlockSpec((NSB, 1), lambda i: (i, 0))],
        out_specs=[pl.BlockSpec((B, NSB), lambda i: (0, i)),
                   pl.BlockSpec((1, B, NSB // CHUNK), lambda i: (i, 0, 0))],
        out_shape=[jax.ShapeDtypeStruct((B, NS), jnp.float32),
                   jax.ShapeDtypeStruct((NS // NSB, B, NSB // CHUNK),
                                        jnp.float32)],
    )(q, semantic_keys, sk_norm)
    m = jnp.transpose(m, (1, 0, 2)).reshape(B, NCHUNK)

    tau = jnp.min(jnp.max(m.reshape(B, TOP_K, NCHUNK // TOP_K), axis=-1),
                  axis=-1)
    tau_b = jnp.broadcast_to(tau[:, None], (B, 16))

    sem_chunks = semantic_scores.reshape(B * NCHUNK, CHUNK)
    ci, cs, cnt = _run_select(sem_chunks, working_scores, m, tau_b)

    counts = cnt[:, 0]

    def _fast(_):
        top_scores, pos = lax.top_k(cs, TOP_K)
        top_indices = jnp.take_along_axis(ci, pos, axis=1)
        return top_scores, top_indices

    def _slow(_):
        scores = jnp.concatenate([working_scores, semantic_scores], axis=-1)
        ts, ti = lax.top_k(scores, TOP_K)
        return ts, ti

    ok = jnp.all(counts <= CAND)
    top_scores, top_indices = lax.cond(ok, _fast, _slow, operand=None)

    weights = jax.nn.softmax(top_scores, axis=-1)
    is_working = top_indices < NW
    w_idx = jnp.clip(top_indices, 0, NW - 1)
    s_idx = jnp.clip(top_indices - NW, 0, NS - 1)
    w_sel = jnp.take_along_axis(working_values, w_idx[..., None], axis=1)
    s_sel = jnp.take(semantic_values, s_idx, axis=0)
    selected_values = jnp.where(is_working[..., None], w_sel, s_sel)
    memory_read = jnp.sum(selected_values * weights[..., None], axis=1)

    aux = {
        "top_indices": top_indices,
        "top_scores": top_scores,
        "weights": weights,
        "working_ratio": jnp.mean((top_indices < NW).astype(jnp.float32)),
    }
    return memory_read, aux
